# baseline (speedup 1.0000x reference)
import functools
import math

import jax
import jax.numpy as jnp
from jax.experimental import pallas as pl
from jax.experimental.pallas import tpu as pltpu

_HEAD = 16
_KG = 64
_TOPK = 32
_DH = 64
_BLK = 256
_NEG = -3.0e38


def _norm_proj_kernel(scal_ref, x_ref, rs_ref, nw_ref, wq_ref, wk_ref, wv_ref,
                      wg_ref, xn_ref, q_ref, k_ref, v_ref, g_ref):
    sg = scal_ref[0]
    y = x_ref[...] * sg
    xn = y * rs_ref[...] * nw_ref[...]
    xn_ref[...] = xn
    q_ref[...] = jnp.dot(xn, wq_ref[...], preferred_element_type=jnp.float32)
    k_ref[...] = jnp.dot(xn, wk_ref[...], preferred_element_type=jnp.float32)
    v_ref[...] = jnp.dot(xn, wv_ref[...], preferred_element_type=jnp.float32)
    g_ref[...] = jnp.dot(xn, wg_ref[...], preferred_element_type=jnp.float32)


def _ffn_kernel(scal_ref, xn_ref, w1_ref, b1_ref, w2_ref, b2_ref, f_ref):
    sa = scal_ref[1]
    h = jnp.dot(xn_ref[...], w1_ref[...], preferred_element_type=jnp.float32)
    h = jax.nn.gelu(h + b1_ref[...])
    f_ref[...] = sa * (
        jnp.dot(h, w2_ref[...], preferred_element_type=jnp.float32) + b2_ref[...])


def _attn_kernel(scal_ref, q_ref, k_ref, v_ref, g_ref, gq_ref, bias_ref, f_ref,
                 wo_ref, out_ref, idx_ref, sc_ref, sw_ref, at_ref, *, n):
    sb = scal_ref[2]
    inv_dh = 1.0 / math.sqrt(_DH)
    inv_kg = 1.0 / math.sqrt(_KG)
    adj = jax.lax.dot_general(
        gq_ref[...], g_ref[...], (((1,), (1,)), ((), ())),
        preferred_element_type=jnp.float32) * inv_kg
    iota = jax.lax.broadcasted_iota(jnp.int32, (_BLK, n), 1)
    tcol = jax.lax.broadcasted_iota(jnp.int32, (_BLK, _TOPK), 1)
    for h in range(_HEAD):
        qh = q_ref[:, h * _DH:(h + 1) * _DH]
        kh = k_ref[:, h * _DH:(h + 1) * _DH]
        s = (jax.lax.dot_general(
            qh, kh, (((1,), (1,)), ((), ())),
            preferred_element_type=jnp.float32) * inv_dh
             + bias_ref[...]) + adj
        sc_ref[...] = s
        sw_ref[...] = s
        m0 = jnp.max(s, axis=-1, keepdims=True)

        def _step(t, idxs):
            swv = sw_ref[...]
            sel = jnp.argmax(swv, axis=-1).astype(jnp.int32).reshape(_BLK, 1)
            pos = iota == sel
            sw_ref[...] = jnp.where(pos, _NEG, swv)
            return jnp.where(tcol == t, sel, idxs)

        idxs = jax.lax.fori_loop(
            0, _TOPK, _step, jnp.zeros((_BLK, _TOPK), jnp.int32))
        idx_ref[h] = idxs
        p = jnp.where(sw_ref[...] == _NEG, jnp.exp(sc_ref[...] - m0), 0.0)
        denom = jnp.sum(p, axis=-1, keepdims=True)
        vh = v_ref[:, h * _DH:(h + 1) * _DH]
        av = jax.lax.dot_general(p, vh, (((1,), (0,)), ((), ())),
                                 preferred_element_type=jnp.float32)
        at_ref[:, h * _DH:(h + 1) * _DH] = av / denom
    out_ref[...] = f_ref[...] + sb * jnp.dot(
        at_ref[...], wo_ref[...], preferred_element_type=jnp.float32)


def _run_proj(scal, x2, rs, nw2, Wq, Wk, Wv, Wg):
    n, d = x2.shape
    kg = Wg.shape[1]
    nblk = n // _BLK
    smem_spec = pl.BlockSpec(memory_space=pltpu.SMEM)
    full = lambda shape: pl.BlockSpec(shape, lambda i: (0,) * len(shape))
    rowblk = lambda w: pl.BlockSpec((_BLK, w), lambda i: (i, 0))
    return pl.pallas_call(
        _norm_proj_kernel,
        grid=(nblk,),
        in_specs=[smem_spec, rowblk(d), rowblk(1), full((1, d)), full((d, d)),
                  full((d, d)), full((d, d)), full((d, kg))],
        out_specs=[rowblk(d), rowblk(d), rowblk(d), rowblk(d), rowblk(kg)],
        out_shape=[jax.ShapeDtypeStruct((n, d), jnp.float32)] * 4
        + [jax.ShapeDtypeStruct((n, kg), jnp.float32)],
        compiler_params=pltpu.CompilerParams(
            dimension_semantics=("parallel",)),
    )(scal, x2, rs, nw2, Wq, Wk, Wv, Wg)


def kernel(x, bias, norm_w, Wq, Wk, Wv, Wo, Wg, W1, b1, W2, b2,
           alpha, beta, gamma, delta):
    b, n, d = x.shape
    hid = W1.shape[1]
    kg = Wg.shape[1]
    nblk = n // _BLK
    x2 = x.reshape(n, d)
    bias2 = bias.reshape(n, n)
    scal = jnp.stack([jax.nn.sigmoid(gamma), jax.nn.sigmoid(alpha),
                      jax.nn.sigmoid(beta), jnp.float32(0.0)])
    y = jax.nn.sigmoid(gamma) * x2
    rs = jax.lax.rsqrt(jnp.mean(y * y, axis=-1, keepdims=True) + 1e-6)
    smem_spec = pl.BlockSpec(memory_space=pltpu.SMEM)
    full = lambda shape: pl.BlockSpec(shape, lambda i: (0,) * len(shape))
    rowblk = lambda w: pl.BlockSpec((_BLK, w), lambda i: (i, 0))

    xn, q, k, v, g = _run_proj(scal, x2, rs, norm_w.reshape(1, d),
                               Wq, Wk, Wv, Wg)

    f = pl.pallas_call(
        _ffn_kernel,
        grid=(nblk,),
        in_specs=[smem_spec, rowblk(d), full((d, hid)), full((1, hid)),
                  full((hid, d)), full((1, d))],
        out_specs=rowblk(d),
        out_shape=jax.ShapeDtypeStruct((n, d), jnp.float32),
        compiler_params=pltpu.CompilerParams(
            dimension_semantics=("parallel",)),
    )(scal, xn, W1, b1.reshape(1, hid), W2, b2.reshape(1, d))

    out2, idx3 = pl.pallas_call(
        functools.partial(_attn_kernel, n=n),
        grid=(nblk,),
        in_specs=[smem_spec, rowblk(d), full((n, d)), full((n, d)),
                  full((n, kg)), rowblk(kg), rowblk(n), rowblk(d),
                  full((d, d))],
        out_specs=[rowblk(d),
                   pl.BlockSpec((_HEAD, _BLK, _TOPK), lambda i: (0, i, 0))],
        out_shape=[jax.ShapeDtypeStruct((n, d), jnp.float32),
                   jax.ShapeDtypeStruct((_HEAD, n, _TOPK), jnp.int32)],
        scratch_shapes=[pltpu.VMEM((_BLK, n), jnp.float32),
                        pltpu.VMEM((_BLK, n), jnp.float32),
                        pltpu.VMEM((_BLK, d), jnp.float32)],
        compiler_params=pltpu.CompilerParams(
            dimension_semantics=("parallel",)),
    )(scal, q, k, v, g, g, bias2, f, Wo)

    return out2.reshape(b, n, d), idx3[None]


# speedup vs baseline: 1.1157x; 1.1157x over previous
import functools
import math

import jax
import jax.numpy as jnp
from jax.experimental import pallas as pl
from jax.experimental.pallas import tpu as pltpu

_HEAD = 16
_KG = 64
_TOPK = 32
_DH = 64
_BLK = 256
_NEG = -3.0e38


def _norm_proj_kernel(scal_ref, x_ref, rs_ref, nw_ref, wq_ref, wk_ref, wv_ref,
                      wg_ref, xn_ref, q_ref, k_ref, v_ref, g_ref):
    sg = scal_ref[0]
    y = x_ref[...] * sg
    xn = y * rs_ref[...] * nw_ref[...]
    xn_ref[...] = xn
    q_ref[...] = jnp.dot(xn, wq_ref[...], preferred_element_type=jnp.float32)
    k_ref[...] = jnp.dot(xn, wk_ref[...], preferred_element_type=jnp.float32)
    v_ref[...] = jnp.dot(xn, wv_ref[...], preferred_element_type=jnp.float32)
    g_ref[...] = jnp.dot(xn, wg_ref[...], preferred_element_type=jnp.float32)


def _ffn_kernel(scal_ref, xn_ref, w1_ref, b1_ref, w2_ref, b2_ref, f_ref):
    sa = scal_ref[1]
    h = jnp.dot(xn_ref[...], w1_ref[...], preferred_element_type=jnp.float32)
    h = jax.nn.gelu(h + b1_ref[...])
    f_ref[...] = sa * (
        jnp.dot(h, w2_ref[...], preferred_element_type=jnp.float32) + b2_ref[...])


def _attn_kernel(scal_ref, q_ref, k_ref, v_ref, g_ref, gq_ref, bias_ref, f_ref,
                 wo_ref, out_ref, idx_ref, sc_ref, sw_ref, at_ref, *, n):
    sb = scal_ref[2]
    inv_dh = 1.0 / math.sqrt(_DH)
    inv_kg = 1.0 / math.sqrt(_KG)
    adj = jax.lax.dot_general(
        gq_ref[...], g_ref[...], (((1,), (1,)), ((), ())),
        preferred_element_type=jnp.float32) * inv_kg
    iota = jax.lax.broadcasted_iota(jnp.int32, (_BLK, n), 1)
    tcol = jax.lax.broadcasted_iota(jnp.int32, (_BLK, _TOPK), 1)
    for h in range(_HEAD):
        qh = q_ref[:, h * _DH:(h + 1) * _DH]
        kh = k_ref[:, h * _DH:(h + 1) * _DH]
        s = (jax.lax.dot_general(
            qh, kh, (((1,), (1,)), ((), ())),
            preferred_element_type=jnp.float32) * inv_dh
             + bias_ref[...]) + adj
        sc_ref[...] = s
        sw_ref[...] = s
        m0 = jnp.max(s, axis=-1, keepdims=True)

        def _step(t, carry):
            idxs, m = carry
            swv = sw_ref[...]
            sel = jnp.min(jnp.where(swv == m, iota, n), axis=-1, keepdims=True)
            nsw = jnp.where(iota == sel, _NEG, swv)
            sw_ref[...] = nsw
            return (jnp.where(tcol == t, sel, idxs),
                    jnp.max(nsw, axis=-1, keepdims=True))

        idxs, _ = jax.lax.fori_loop(
            0, _TOPK, _step,
            (jnp.zeros((_BLK, _TOPK), jnp.int32), m0))
        idx_ref[h] = idxs
        p = jnp.where(sw_ref[...] == _NEG, jnp.exp(sc_ref[...] - m0), 0.0)
        denom = jnp.sum(p, axis=-1, keepdims=True)
        vh = v_ref[:, h * _DH:(h + 1) * _DH]
        av = jax.lax.dot_general(p, vh, (((1,), (0,)), ((), ())),
                                 preferred_element_type=jnp.float32)
        at_ref[:, h * _DH:(h + 1) * _DH] = av / denom
    out_ref[...] = f_ref[...] + sb * jnp.dot(
        at_ref[...], wo_ref[...], preferred_element_type=jnp.float32)


def _run_proj(scal, x2, rs, nw2, Wq, Wk, Wv, Wg):
    n, d = x2.shape
    kg = Wg.shape[1]
    nblk = n // _BLK
    smem_spec = pl.BlockSpec(memory_space=pltpu.SMEM)
    full = lambda shape: pl.BlockSpec(shape, lambda i: (0,) * len(shape))
    rowblk = lambda w: pl.BlockSpec((_BLK, w), lambda i: (i, 0))
    return pl.pallas_call(
        _norm_proj_kernel,
        grid=(nblk,),
        in_specs=[smem_spec, rowblk(d), rowblk(1), full((1, d)), full((d, d)),
                  full((d, d)), full((d, d)), full((d, kg))],
        out_specs=[rowblk(d), rowblk(d), rowblk(d), rowblk(d), rowblk(kg)],
        out_shape=[jax.ShapeDtypeStruct((n, d), jnp.float32)] * 4
        + [jax.ShapeDtypeStruct((n, kg), jnp.float32)],
        compiler_params=pltpu.CompilerParams(
            dimension_semantics=("parallel",)),
    )(scal, x2, rs, nw2, Wq, Wk, Wv, Wg)


def kernel(x, bias, norm_w, Wq, Wk, Wv, Wo, Wg, W1, b1, W2, b2,
           alpha, beta, gamma, delta):
    b, n, d = x.shape
    hid = W1.shape[1]
    kg = Wg.shape[1]
    nblk = n // _BLK
    x2 = x.reshape(n, d)
    bias2 = bias.reshape(n, n)
    scal = jnp.stack([jax.nn.sigmoid(gamma), jax.nn.sigmoid(alpha),
                      jax.nn.sigmoid(beta), jnp.float32(0.0)])
    y = jax.nn.sigmoid(gamma) * x2
    rs = jax.lax.rsqrt(jnp.mean(y * y, axis=-1, keepdims=True) + 1e-6)
    smem_spec = pl.BlockSpec(memory_space=pltpu.SMEM)
    full = lambda shape: pl.BlockSpec(shape, lambda i: (0,) * len(shape))
    rowblk = lambda w: pl.BlockSpec((_BLK, w), lambda i: (i, 0))

    xn, q, k, v, g = _run_proj(scal, x2, rs, norm_w.reshape(1, d),
                               Wq, Wk, Wv, Wg)

    f = pl.pallas_call(
        _ffn_kernel,
        grid=(nblk,),
        in_specs=[smem_spec, rowblk(d), full((d, hid)), full((1, hid)),
                  full((hid, d)), full((1, d))],
        out_specs=rowblk(d),
        out_shape=jax.ShapeDtypeStruct((n, d), jnp.float32),
        compiler_params=pltpu.CompilerParams(
            dimension_semantics=("parallel",)),
    )(scal, xn, W1, b1.reshape(1, hid), W2, b2.reshape(1, d))

    out2, idx3 = pl.pallas_call(
        functools.partial(_attn_kernel, n=n),
        grid=(nblk,),
        in_specs=[smem_spec, rowblk(d), full((n, d)), full((n, d)),
                  full((n, kg)), rowblk(kg), rowblk(n), rowblk(d),
                  full((d, d))],
        out_specs=[rowblk(d),
                   pl.BlockSpec((_HEAD, _BLK, _TOPK), lambda i: (0, i, 0))],
        out_shape=[jax.ShapeDtypeStruct((n, d), jnp.float32),
                   jax.ShapeDtypeStruct((_HEAD, n, _TOPK), jnp.int32)],
        scratch_shapes=[pltpu.VMEM((_BLK, n), jnp.float32),
                        pltpu.VMEM((_BLK, n), jnp.float32),
                        pltpu.VMEM((_BLK, d), jnp.float32)],
        compiler_params=pltpu.CompilerParams(
            dimension_semantics=("parallel",)),
    )(scal, q, k, v, g, g, bias2, f, Wo)

    return out2.reshape(b, n, d), idx3[None]


# speedup vs baseline: 1.1520x; 1.0326x over previous
import functools
import math

import jax
import jax.numpy as jnp
from jax.experimental import pallas as pl
from jax.experimental.pallas import tpu as pltpu

_HEAD = 16
_KG = 64
_TOPK = 32
_DH = 64
_BLK = 256
_NEG = -3.0e38


def _norm_proj_kernel(scal_ref, x_ref, rs_ref, nw_ref, wq_ref, wk_ref, wv_ref,
                      wg_ref, xn_ref, q_ref, k_ref, v_ref, g_ref):
    sg = scal_ref[0]
    y = x_ref[...] * sg
    xn = y * rs_ref[...] * nw_ref[...]
    xn_ref[...] = xn
    q_ref[...] = jnp.dot(xn, wq_ref[...], preferred_element_type=jnp.float32)
    k_ref[...] = jnp.dot(xn, wk_ref[...], preferred_element_type=jnp.float32)
    v_ref[...] = jnp.dot(xn, wv_ref[...], preferred_element_type=jnp.float32)
    g_ref[...] = jnp.dot(xn, wg_ref[...], preferred_element_type=jnp.float32)


def _ffn_kernel(scal_ref, xn_ref, w1_ref, b1_ref, w2_ref, b2_ref, f_ref):
    sa = scal_ref[1]
    h = jnp.dot(xn_ref[...], w1_ref[...], preferred_element_type=jnp.float32)
    h = jax.nn.gelu(h + b1_ref[...])
    f_ref[...] = sa * (
        jnp.dot(h, w2_ref[...], preferred_element_type=jnp.float32) + b2_ref[...])


def _attn_kernel(scal_ref, q_ref, k_ref, v_ref, g_ref, gq_ref, bias_ref, f_ref,
                 wo_ref, out_ref, idx_ref, sc_ref, sw_ref, at_ref, *, n):
    sb = scal_ref[2]
    inv_dh = 1.0 / math.sqrt(_DH)
    inv_kg = 1.0 / math.sqrt(_KG)
    adj = jax.lax.dot_general(
        gq_ref[...], g_ref[...], (((1,), (1,)), ((), ())),
        preferred_element_type=jnp.float32) * inv_kg
    iota = jax.lax.broadcasted_iota(jnp.int32, (_BLK, n), 1)
    tcol = jax.lax.broadcasted_iota(jnp.int32, (_BLK, _TOPK), 1)
    for h in range(_HEAD):
        qh = q_ref[:, h * _DH:(h + 1) * _DH]
        kh = k_ref[:, h * _DH:(h + 1) * _DH]
        s = (jax.lax.dot_general(
            qh, kh, (((1,), (1,)), ((), ())),
            preferred_element_type=jnp.float32) * inv_dh
             + bias_ref[...]) + adj
        sc_ref[...] = s
        sw_ref[...] = s
        m0 = jnp.max(s, axis=-1, keepdims=True)

        def _step(t, idxs):
            swv = sw_ref[...]
            m = jnp.max(swv, axis=-1, keepdims=True)
            sel = jnp.min(jnp.where(swv == m, iota, n), axis=-1, keepdims=True)
            pos = iota == sel
            sw_ref[...] = jnp.where(pos, _NEG, swv)
            return jnp.where(tcol == t, sel, idxs)

        idxs = jax.lax.fori_loop(
            0, _TOPK, _step, jnp.zeros((_BLK, _TOPK), jnp.int32))
        idx_ref[h] = idxs
        p = jnp.where(sw_ref[...] == _NEG, jnp.exp(sc_ref[...] - m0), 0.0)
        denom = jnp.sum(p, axis=-1, keepdims=True)
        vh = v_ref[:, h * _DH:(h + 1) * _DH]
        av = jax.lax.dot_general(p, vh, (((1,), (0,)), ((), ())),
                                 preferred_element_type=jnp.float32)
        at_ref[:, h * _DH:(h + 1) * _DH] = av / denom
    out_ref[...] = f_ref[...] + sb * jnp.dot(
        at_ref[...], wo_ref[...], preferred_element_type=jnp.float32)


def _run_proj(scal, x2, rs, nw2, Wq, Wk, Wv, Wg):
    n, d = x2.shape
    kg = Wg.shape[1]
    nblk = n // _BLK
    smem_spec = pl.BlockSpec(memory_space=pltpu.SMEM)
    full = lambda shape: pl.BlockSpec(shape, lambda i: (0,) * len(shape))
    rowblk = lambda w: pl.BlockSpec((_BLK, w), lambda i: (i, 0))
    return pl.pallas_call(
        _norm_proj_kernel,
        grid=(nblk,),
        in_specs=[smem_spec, rowblk(d), rowblk(1), full((1, d)), full((d, d)),
                  full((d, d)), full((d, d)), full((d, kg))],
        out_specs=[rowblk(d), rowblk(d), rowblk(d), rowblk(d), rowblk(kg)],
        out_shape=[jax.ShapeDtypeStruct((n, d), jnp.float32)] * 4
        + [jax.ShapeDtypeStruct((n, kg), jnp.float32)],
        compiler_params=pltpu.CompilerParams(
            dimension_semantics=("parallel",)),
    )(scal, x2, rs, nw2, Wq, Wk, Wv, Wg)


def kernel(x, bias, norm_w, Wq, Wk, Wv, Wo, Wg, W1, b1, W2, b2,
           alpha, beta, gamma, delta):
    b, n, d = x.shape
    hid = W1.shape[1]
    kg = Wg.shape[1]
    nblk = n // _BLK
    x2 = x.reshape(n, d)
    bias2 = bias.reshape(n, n)
    scal = jnp.stack([jax.nn.sigmoid(gamma), jax.nn.sigmoid(alpha),
                      jax.nn.sigmoid(beta), jnp.float32(0.0)])
    y = jax.nn.sigmoid(gamma) * x2
    rs = jax.lax.rsqrt(jnp.mean(y * y, axis=-1, keepdims=True) + 1e-6)
    smem_spec = pl.BlockSpec(memory_space=pltpu.SMEM)
    full = lambda shape: pl.BlockSpec(shape, lambda i: (0,) * len(shape))
    rowblk = lambda w: pl.BlockSpec((_BLK, w), lambda i: (i, 0))

    xn, q, k, v, g = _run_proj(scal, x2, rs, norm_w.reshape(1, d),
                               Wq, Wk, Wv, Wg)

    f = pl.pallas_call(
        _ffn_kernel,
        grid=(nblk,),
        in_specs=[smem_spec, rowblk(d), full((d, hid)), full((1, hid)),
                  full((hid, d)), full((1, d))],
        out_specs=rowblk(d),
        out_shape=jax.ShapeDtypeStruct((n, d), jnp.float32),
        compiler_params=pltpu.CompilerParams(
            dimension_semantics=("parallel",)),
    )(scal, xn, W1, b1.reshape(1, hid), W2, b2.reshape(1, d))

    out2, idx3 = pl.pallas_call(
        functools.partial(_attn_kernel, n=n),
        grid=(nblk,),
        in_specs=[smem_spec, rowblk(d), full((n, d)), full((n, d)),
                  full((n, kg)), rowblk(kg), rowblk(n), rowblk(d),
                  full((d, d))],
        out_specs=[rowblk(d),
                   pl.BlockSpec((_HEAD, _BLK, _TOPK), lambda i: (0, i, 0))],
        out_shape=[jax.ShapeDtypeStruct((n, d), jnp.float32),
                   jax.ShapeDtypeStruct((_HEAD, n, _TOPK), jnp.int32)],
        scratch_shapes=[pltpu.VMEM((_BLK, n), jnp.float32),
                        pltpu.VMEM((_BLK, n), jnp.float32),
                        pltpu.VMEM((_BLK, d), jnp.float32)],
        compiler_params=pltpu.CompilerParams(
            dimension_semantics=("parallel",)),
    )(scal, q, k, v, g, g, bias2, f, Wo)

    return out2.reshape(b, n, d), idx3[None]


# speedup vs baseline: 1.2469x; 1.0824x over previous
import functools
import math

import jax
import jax.numpy as jnp
from jax.experimental import pallas as pl
from jax.experimental.pallas import tpu as pltpu

_HEAD = 16
_KG = 64
_TOPK = 32
_DH = 64
_BLK = 256
_NEG = -3.0e38


def _norm_proj_kernel(scal_ref, x_ref, rs_ref, nw_ref, wq_ref, wk_ref, wv_ref,
                      wg_ref, xn_ref, q_ref, k_ref, v_ref, g_ref):
    sg = scal_ref[0]
    y = x_ref[...] * sg
    xn = y * rs_ref[...] * nw_ref[...]
    xn_ref[...] = xn
    q_ref[...] = jnp.dot(xn, wq_ref[...], preferred_element_type=jnp.float32)
    k_ref[...] = jnp.dot(xn, wk_ref[...], preferred_element_type=jnp.float32)
    v_ref[...] = jnp.dot(xn, wv_ref[...], preferred_element_type=jnp.float32)
    g_ref[...] = jnp.dot(xn, wg_ref[...], preferred_element_type=jnp.float32)


def _ffn_kernel(scal_ref, xn_ref, w1_ref, b1_ref, w2_ref, b2_ref, f_ref):
    sa = scal_ref[1]
    h = jnp.dot(xn_ref[...], w1_ref[...], preferred_element_type=jnp.float32)
    h = jax.nn.gelu(h + b1_ref[...])
    f_ref[...] = sa * (
        jnp.dot(h, w2_ref[...], preferred_element_type=jnp.float32) + b2_ref[...])


def _attn_kernel(scal_ref, q_ref, k_ref, v_ref, g_ref, gq_ref, bias_ref, f_ref,
                 wo_ref, out_ref, idx_ref, sc_ref, sw_ref, sc2_ref, sw2_ref,
                 at_ref, *, n):
    sb = scal_ref[2]
    inv_dh = 1.0 / math.sqrt(_DH)
    inv_kg = 1.0 / math.sqrt(_KG)
    adj = jax.lax.dot_general(
        gq_ref[...], g_ref[...], (((1,), (1,)), ((), ())),
        preferred_element_type=jnp.float32) * inv_kg
    iota = jax.lax.broadcasted_iota(jnp.int32, (_BLK, n), 1)
    tcol = jax.lax.broadcasted_iota(jnp.int32, (_BLK, _TOPK), 1)

    def _scores(h):
        qh = q_ref[:, h * _DH:(h + 1) * _DH]
        kh = k_ref[:, h * _DH:(h + 1) * _DH]
        return (jax.lax.dot_general(
            qh, kh, (((1,), (1,)), ((), ())),
            preferred_element_type=jnp.float32) * inv_dh
            + bias_ref[...]) + adj

    def _finish(h, scr, swr, m0):
        p = jnp.where(swr[...] == _NEG, jnp.exp(scr[...] - m0), 0.0)
        denom = jnp.sum(p, axis=-1, keepdims=True)
        vh = v_ref[:, h * _DH:(h + 1) * _DH]
        av = jax.lax.dot_general(p, vh, (((1,), (0,)), ((), ())),
                                 preferred_element_type=jnp.float32)
        at_ref[:, h * _DH:(h + 1) * _DH] = av / denom

    for ha in range(0, _HEAD, 2):
        hb = ha + 1
        sa = _scores(ha)
        sc_ref[...] = sa
        sw_ref[...] = sa
        m0a = jnp.max(sa, axis=-1, keepdims=True)
        sb_ = _scores(hb)
        sc2_ref[...] = sb_
        sw2_ref[...] = sb_
        m0b = jnp.max(sb_, axis=-1, keepdims=True)

        def _step(t, carry):
            ia, ib = carry
            swa = sw_ref[...]
            swb = sw2_ref[...]
            ma = jnp.max(swa, axis=-1, keepdims=True)
            mb = jnp.max(swb, axis=-1, keepdims=True)
            sela = jnp.min(jnp.where(swa == ma, iota, n), axis=-1,
                           keepdims=True)
            selb = jnp.min(jnp.where(swb == mb, iota, n), axis=-1,
                           keepdims=True)
            sw_ref[...] = jnp.where(iota == sela, _NEG, swa)
            sw2_ref[...] = jnp.where(iota == selb, _NEG, swb)
            return (jnp.where(tcol == t, sela, ia),
                    jnp.where(tcol == t, selb, ib))

        za = jnp.zeros((_BLK, _TOPK), jnp.int32)
        ia, ib = jax.lax.fori_loop(0, _TOPK, _step, (za, za))
        idx_ref[ha] = ia
        idx_ref[hb] = ib
        _finish(ha, sc_ref, sw_ref, m0a)
        _finish(hb, sc2_ref, sw2_ref, m0b)
    out_ref[...] = f_ref[...] + sb * jnp.dot(
        at_ref[...], wo_ref[...], preferred_element_type=jnp.float32)


def _run_proj(scal, x2, rs, nw2, Wq, Wk, Wv, Wg):
    n, d = x2.shape
    kg = Wg.shape[1]
    nblk = n // _BLK
    smem_spec = pl.BlockSpec(memory_space=pltpu.SMEM)
    full = lambda shape: pl.BlockSpec(shape, lambda i: (0,) * len(shape))
    rowblk = lambda w: pl.BlockSpec((_BLK, w), lambda i: (i, 0))
    return pl.pallas_call(
        _norm_proj_kernel,
        grid=(nblk,),
        in_specs=[smem_spec, rowblk(d), rowblk(1), full((1, d)), full((d, d)),
                  full((d, d)), full((d, d)), full((d, kg))],
        out_specs=[rowblk(d), rowblk(d), rowblk(d), rowblk(d), rowblk(kg)],
        out_shape=[jax.ShapeDtypeStruct((n, d), jnp.float32)] * 4
        + [jax.ShapeDtypeStruct((n, kg), jnp.float32)],
        compiler_params=pltpu.CompilerParams(
            dimension_semantics=("parallel",)),
    )(scal, x2, rs, nw2, Wq, Wk, Wv, Wg)


def kernel(x, bias, norm_w, Wq, Wk, Wv, Wo, Wg, W1, b1, W2, b2,
           alpha, beta, gamma, delta):
    b, n, d = x.shape
    hid = W1.shape[1]
    kg = Wg.shape[1]
    nblk = n // _BLK
    x2 = x.reshape(n, d)
    bias2 = bias.reshape(n, n)
    scal = jnp.stack([jax.nn.sigmoid(gamma), jax.nn.sigmoid(alpha),
                      jax.nn.sigmoid(beta), jnp.float32(0.0)])
    y = jax.nn.sigmoid(gamma) * x2
    rs = jax.lax.rsqrt(jnp.mean(y * y, axis=-1, keepdims=True) + 1e-6)
    smem_spec = pl.BlockSpec(memory_space=pltpu.SMEM)
    full = lambda shape: pl.BlockSpec(shape, lambda i: (0,) * len(shape))
    rowblk = lambda w: pl.BlockSpec((_BLK, w), lambda i: (i, 0))

    xn, q, k, v, g = _run_proj(scal, x2, rs, norm_w.reshape(1, d),
                               Wq, Wk, Wv, Wg)

    f = pl.pallas_call(
        _ffn_kernel,
        grid=(nblk,),
        in_specs=[smem_spec, rowblk(d), full((d, hid)), full((1, hid)),
                  full((hid, d)), full((1, d))],
        out_specs=rowblk(d),
        out_shape=jax.ShapeDtypeStruct((n, d), jnp.float32),
        compiler_params=pltpu.CompilerParams(
            dimension_semantics=("parallel",)),
    )(scal, xn, W1, b1.reshape(1, hid), W2, b2.reshape(1, d))

    out2, idx3 = pl.pallas_call(
        functools.partial(_attn_kernel, n=n),
        grid=(nblk,),
        in_specs=[smem_spec, rowblk(d), full((n, d)), full((n, d)),
                  full((n, kg)), rowblk(kg), rowblk(n), rowblk(d),
                  full((d, d))],
        out_specs=[rowblk(d),
                   pl.BlockSpec((_HEAD, _BLK, _TOPK), lambda i: (0, i, 0))],
        out_shape=[jax.ShapeDtypeStruct((n, d), jnp.float32),
                   jax.ShapeDtypeStruct((_HEAD, n, _TOPK), jnp.int32)],
        scratch_shapes=[pltpu.VMEM((_BLK, n), jnp.float32),
                        pltpu.VMEM((_BLK, n), jnp.float32),
                        pltpu.VMEM((_BLK, n), jnp.float32),
                        pltpu.VMEM((_BLK, n), jnp.float32),
                        pltpu.VMEM((_BLK, d), jnp.float32)],
        compiler_params=pltpu.CompilerParams(
            dimension_semantics=("parallel",)),
    )(scal, q, k, v, g, g, bias2, f, Wo)

    return out2.reshape(b, n, d), idx3[None]


# speedup vs baseline: 1.2492x; 1.0018x over previous
import functools
import math

import jax
import jax.numpy as jnp
from jax.experimental import pallas as pl
from jax.experimental.pallas import tpu as pltpu

_HEAD = 16
_KG = 64
_TOPK = 32
_DH = 64
_BLK = 256
_ILV = 4
_NEG = -3.0e38


def _norm_proj_kernel(scal_ref, x_ref, rs_ref, nw_ref, wq_ref, wk_ref, wv_ref,
                      wg_ref, xn_ref, q_ref, k_ref, v_ref, g_ref):
    sg = scal_ref[0]
    y = x_ref[...] * sg
    xn = y * rs_ref[...] * nw_ref[...]
    xn_ref[...] = xn
    q_ref[...] = jnp.dot(xn, wq_ref[...], preferred_element_type=jnp.float32)
    k_ref[...] = jnp.dot(xn, wk_ref[...], preferred_element_type=jnp.float32)
    v_ref[...] = jnp.dot(xn, wv_ref[...], preferred_element_type=jnp.float32)
    g_ref[...] = jnp.dot(xn, wg_ref[...], preferred_element_type=jnp.float32)


def _ffn_kernel(scal_ref, xn_ref, w1_ref, b1_ref, w2_ref, b2_ref, f_ref):
    sa = scal_ref[1]
    h = jnp.dot(xn_ref[...], w1_ref[...], preferred_element_type=jnp.float32)
    h = jax.nn.gelu(h + b1_ref[...])
    f_ref[...] = sa * (
        jnp.dot(h, w2_ref[...], preferred_element_type=jnp.float32) + b2_ref[...])


def _attn_kernel(scal_ref, q_ref, k_ref, v_ref, g_ref, gq_ref, bias_ref, f_ref,
                 wo_ref, out_ref, idx_ref, *refs, n):
    at_ref = refs[-1]
    sws = refs[0:_ILV]
    sb = scal_ref[2]
    inv_dh = 1.0 / math.sqrt(_DH)
    inv_kg = 1.0 / math.sqrt(_KG)
    adj = jax.lax.dot_general(
        gq_ref[...], g_ref[...], (((1,), (1,)), ((), ())),
        preferred_element_type=jnp.float32) * inv_kg
    iota = jax.lax.broadcasted_iota(jnp.int32, (_BLK, n), 1)
    tcol = jax.lax.broadcasted_iota(jnp.int32, (_BLK, _TOPK), 1)

    def _scores(h):
        qh = q_ref[:, h * _DH:(h + 1) * _DH]
        kh = k_ref[:, h * _DH:(h + 1) * _DH]
        return (jax.lax.dot_general(
            qh, kh, (((1,), (1,)), ((), ())),
            preferred_element_type=jnp.float32) * inv_dh
            + bias_ref[...]) + adj

    def _finish(h, swr, m0):
        p = jnp.where(swr[...] == _NEG, jnp.exp(_scores(h) - m0), 0.0)
        denom = jnp.sum(p, axis=-1, keepdims=True)
        vh = v_ref[:, h * _DH:(h + 1) * _DH]
        av = jax.lax.dot_general(p, vh, (((1,), (0,)), ((), ())),
                                 preferred_element_type=jnp.float32)
        at_ref[:, h * _DH:(h + 1) * _DH] = av / denom

    for ha in range(0, _HEAD, _ILV):
        hs = list(range(ha, ha + _ILV))
        m0s = []
        for j, h in enumerate(hs):
            s = _scores(h)
            sws[j][...] = s
            m0s.append(jnp.max(s, axis=-1, keepdims=True))

        def _step(t, carry):
            out = []
            for j in range(_ILV):
                sw = sws[j][...]
                m = jnp.max(sw, axis=-1, keepdims=True)
                sel = jnp.min(jnp.where(sw == m, iota, n), axis=-1,
                              keepdims=True)
                sws[j][...] = jnp.where(iota == sel, _NEG, sw)
                out.append(jnp.where(tcol == t, sel, carry[j]))
            return tuple(out)

        za = jnp.zeros((_BLK, _TOPK), jnp.int32)
        idxs = jax.lax.fori_loop(0, _TOPK, _step, (za,) * _ILV)
        for j, h in enumerate(hs):
            idx_ref[h] = idxs[j]
            _finish(h, sws[j], m0s[j])
    out_ref[...] = f_ref[...] + sb * jnp.dot(
        at_ref[...], wo_ref[...], preferred_element_type=jnp.float32)


def _run_proj(scal, x2, rs, nw2, Wq, Wk, Wv, Wg):
    n, d = x2.shape
    kg = Wg.shape[1]
    nblk = n // _BLK
    smem_spec = pl.BlockSpec(memory_space=pltpu.SMEM)
    full = lambda shape: pl.BlockSpec(shape, lambda i: (0,) * len(shape))
    rowblk = lambda w: pl.BlockSpec((_BLK, w), lambda i: (i, 0))
    return pl.pallas_call(
        _norm_proj_kernel,
        grid=(nblk,),
        in_specs=[smem_spec, rowblk(d), rowblk(1), full((1, d)), full((d, d)),
                  full((d, d)), full((d, d)), full((d, kg))],
        out_specs=[rowblk(d), rowblk(d), rowblk(d), rowblk(d), rowblk(kg)],
        out_shape=[jax.ShapeDtypeStruct((n, d), jnp.float32)] * 4
        + [jax.ShapeDtypeStruct((n, kg), jnp.float32)],
        compiler_params=pltpu.CompilerParams(
            dimension_semantics=("parallel",)),
    )(scal, x2, rs, nw2, Wq, Wk, Wv, Wg)


def kernel(x, bias, norm_w, Wq, Wk, Wv, Wo, Wg, W1, b1, W2, b2,
           alpha, beta, gamma, delta):
    b, n, d = x.shape
    hid = W1.shape[1]
    kg = Wg.shape[1]
    nblk = n // _BLK
    x2 = x.reshape(n, d)
    bias2 = bias.reshape(n, n)
    scal = jnp.stack([jax.nn.sigmoid(gamma), jax.nn.sigmoid(alpha),
                      jax.nn.sigmoid(beta), jnp.float32(0.0)])
    y = jax.nn.sigmoid(gamma) * x2
    rs = jax.lax.rsqrt(jnp.mean(y * y, axis=-1, keepdims=True) + 1e-6)
    smem_spec = pl.BlockSpec(memory_space=pltpu.SMEM)
    full = lambda shape: pl.BlockSpec(shape, lambda i: (0,) * len(shape))
    rowblk = lambda w: pl.BlockSpec((_BLK, w), lambda i: (i, 0))

    xn, q, k, v, g = _run_proj(scal, x2, rs, norm_w.reshape(1, d),
                               Wq, Wk, Wv, Wg)

    f = pl.pallas_call(
        _ffn_kernel,
        grid=(nblk,),
        in_specs=[smem_spec, rowblk(d), full((d, hid)), full((1, hid)),
                  full((hid, d)), full((1, d))],
        out_specs=rowblk(d),
        out_shape=jax.ShapeDtypeStruct((n, d), jnp.float32),
        compiler_params=pltpu.CompilerParams(
            dimension_semantics=("parallel",)),
    )(scal, xn, W1, b1.reshape(1, hid), W2, b2.reshape(1, d))

    out2, idx3 = pl.pallas_call(
        functools.partial(_attn_kernel, n=n),
        grid=(nblk,),
        in_specs=[smem_spec, rowblk(d), full((n, d)), full((n, d)),
                  full((n, kg)), rowblk(kg), rowblk(n), rowblk(d),
                  full((d, d))],
        out_specs=[rowblk(d),
                   pl.BlockSpec((_HEAD, _BLK, _TOPK), lambda i: (0, i, 0))],
        out_shape=[jax.ShapeDtypeStruct((n, d), jnp.float32),
                   jax.ShapeDtypeStruct((_HEAD, n, _TOPK), jnp.int32)],
        scratch_shapes=[pltpu.VMEM((_BLK, n), jnp.float32)] * _ILV
        + [pltpu.VMEM((_BLK, d), jnp.float32)],
        compiler_params=pltpu.CompilerParams(
            dimension_semantics=("parallel",)),
    )(scal, q, k, v, g, g, bias2, f, Wo)

    return out2.reshape(b, n, d), idx3[None]
